# baseline (speedup 1.0000x reference)
import jax
import jax.numpy as jnp
from jax import lax
from jax.experimental import pallas as pl
from jax.experimental.pallas import tpu as pltpu
from jax.experimental.pallas import tpu_sc as plsc

NUM_EMB = 256
IN_CH = 96
B = 4
H = 224
W = 224
IMG = H * W
NIMG = B * IN_CH
NTOT = NIMG * IMG
LANES = 16
NWORKERS = 32
HROWS = 112
NH = NIMG * 2
H_PER_W = NH // NWORKERS
NLSL = W // LANES


_RED_ROWS = 24
_RED_MID = NTOT // _RED_ROWS // 128


def _reduce_body(x_ref, scale_ref, sc_ref, sum_ref, sq_ref, mx_ref):
    blk = x_ref[...]
    s = jnp.full((1, 1), jnp.sum(blk), dtype=jnp.float32)
    sq = jnp.full((1, 1), jnp.sum(blk * blk), dtype=jnp.float32)
    m = jnp.full((1, 1), jnp.max(jnp.abs(blk)), dtype=jnp.float32)

    @pl.when(pl.program_id(0) == 0)
    def _():
        zero = jnp.zeros((1, 1), jnp.float32)
        sum_ref[...] = zero
        sq_ref[...] = zero
        mx_ref[...] = zero

    sum_ref[...] += s
    sq_ref[...] += sq
    mx_ref[...] = jnp.maximum(mx_ref[...], m)

    @pl.when(pl.program_id(0) == _RED_ROWS - 1)
    def _():
        n = jnp.float32(NTOT)
        sv = sum_ref[...]
        sqv = sq_ref[...]
        mxv = mx_ref[...]
        var = (sqv - sv * sv / n) / (n - 1.0)
        std = jnp.sqrt(var)
        min_scale = 2.5 * 0.999 + std * 0.001
        max_scale = 3.5 * 0.999 + mxv * 0.001
        eps = 0.1 * (max_scale - min_scale)
        lo = jnp.broadcast_to(min_scale * (1.0 + eps), (8, 128))
        hi = jnp.broadcast_to(max_scale * (1.0 - eps), (8, 128))
        sc_ref[...] = jnp.minimum(jnp.maximum(scale_ref[...], lo), hi)


def _reductions_sc(x, scale8):
    xr = x.reshape(_RED_ROWS, _RED_MID, 128)
    out = pl.pallas_call(
        _reduce_body,
        grid=(_RED_ROWS,),
        in_specs=[
            pl.BlockSpec((1, _RED_MID, 128), lambda i: (i, 0, 0)),
            pl.BlockSpec((8, 128), lambda i: (0, 0)),
        ],
        out_specs=[
            pl.BlockSpec((8, 128), lambda i: (0, 0)),
            pl.BlockSpec((1, 1), lambda i: (0, 0)),
            pl.BlockSpec((1, 1), lambda i: (0, 0)),
            pl.BlockSpec((1, 1), lambda i: (0, 0)),
        ],
        out_shape=[
            jax.ShapeDtypeStruct((8, 128), jnp.float32),
            jax.ShapeDtypeStruct((1, 1), jnp.float32),
            jax.ShapeDtypeStruct((1, 1), jnp.float32),
            jax.ShapeDtypeStruct((1, 1), jnp.float32),
        ],
    )(xr, scale8)
    return out[0]


def _sc_compute_chunk(xbuf, tab_v, scv, offv):
    @plsc.parallel_loop(0, HROWS, step=1)
    def _(r):
        for t in range(NLSL):
            sl = pl.ds(t * LANES, LANES)
            xv = xbuf[r, sl]
            rr = xv / scv
            rr = jnp.minimum(jnp.maximum(rr, -1.0), 1.0)
            out01 = (rr + 1.0) * 0.5
            out3 = out01 * jnp.float32(NUM_EMB - 1)
            out4 = out3 + offv
            li = out4.astype(jnp.int32)
            lf = li.astype(jnp.float32)
            frac = out4 - lf
            ui = jnp.where(out4 > lf, li + 1, li)
            lv = plsc.load_gather(tab_v, [li])
            uv = plsc.load_gather(tab_v, [ui])
            res = frac * lv + (1.0 - frac) * uv
            xbuf[r, sl] = res


_NBUF = 3


def _sc_body(x_hbm, sc_hbm, emb_hbm, out_hbm, tab_v, scv_v, xb0, xb1, xb2,
             sin0, sin1, sin2, sout0, sout1, sout2):
    wid = lax.axis_index("s") * 2 + lax.axis_index("c")

    pltpu.sync_copy(emb_hbm, tab_v)
    pltpu.sync_copy(sc_hbm.at[0], scv_v)

    bufs = (xb0, xb1, xb2)
    sins = (sin0, sin1, sin2)
    souts = (sout0, sout1, sout2)
    h0 = wid * H_PER_W

    def addr(j):
        hm = h0 + j
        m = lax.div(hm, 2)
        half = lax.rem(hm, 2)
        bb = lax.div(m, IN_CH)
        cc = lax.rem(m, IN_CH)
        return bb, cc, half * HROWS

    def in_copy(j, b):
        bb, cc, r0 = addr(j)
        return pltpu.make_async_copy(
            x_hbm.at[bb, cc, pl.ds(r0, HROWS), :], bufs[b], sins[b])

    def out_copy(j, b):
        bb, cc, r0 = addr(j)
        return pltpu.make_async_copy(
            bufs[b], out_hbm.at[bb, cc, pl.ds(r0, HROWS), :], souts[b])

    in_copy(0, 0).start()
    in_copy(1, 1).start()

    def step(k, _):
        for b in range(_NBUF):
            j = k * _NBUF + b
            c = lax.rem(lax.div(h0 + j, 2), IN_CH)
            cvec = jnp.full((LANES,), c, dtype=jnp.int32)
            scv = plsc.load_gather(scv_v, [cvec])
            offv = jnp.full((LANES,), (c * NUM_EMB).astype(jnp.float32),
                            dtype=jnp.float32)

            in_copy(j, b).wait()
            _sc_compute_chunk(bufs[b], tab_v, scv, offv)
            out_copy(j, b).start()

            pb = (b + _NBUF - 1) % _NBUF
            jn = j + 2
            if b == 0:
                @pl.when(k > 0)
                def _():
                    out_copy(jn - _NBUF, pb).wait()

                in_copy(jn, pb).start()
            else:
                @pl.when(jn < H_PER_W)
                def _():
                    out_copy(jn - _NBUF, pb).wait()
                    in_copy(jn, pb).start()
        return 0

    lax.fori_loop(0, H_PER_W // _NBUF, step, 0)
    for b in range(_NBUF):
        out_copy(H_PER_W - _NBUF + b, b).wait()


def _sc_remap(x, sc8, emb_flat):
    mesh = plsc.VectorSubcoreMesh(core_axis_name="c", subcore_axis_name="s")
    fn = pl.kernel(
        _sc_body,
        out_type=jax.ShapeDtypeStruct((B, IN_CH, H, W), jnp.float32),
        mesh=mesh,
        compiler_params=pltpu.CompilerParams(
            needs_layout_passes=False, use_tc_tiling_on_sc=True),
        scratch_types=[
            pltpu.VMEM((NUM_EMB * IN_CH,), jnp.float32),
            pltpu.VMEM((128,), jnp.float32),
            pltpu.VMEM((HROWS, W), jnp.float32),
            pltpu.VMEM((HROWS, W), jnp.float32),
            pltpu.VMEM((HROWS, W), jnp.float32),
            pltpu.SemaphoreType.DMA,
            pltpu.SemaphoreType.DMA,
            pltpu.SemaphoreType.DMA,
            pltpu.SemaphoreType.DMA,
            pltpu.SemaphoreType.DMA,
            pltpu.SemaphoreType.DMA,
        ],
    )
    return fn(x, sc8, emb_flat)


def kernel(x, scale, emb_weight):
    scale8 = jnp.zeros((8, 128), jnp.float32).at[0, :IN_CH].set(
        scale.reshape(IN_CH))
    sc8 = _reductions_sc(x, scale8)
    return _sc_remap(x, sc8, emb_weight.reshape(NUM_EMB * IN_CH))


# speedup vs baseline: 1.3550x; 1.3550x over previous
import jax
import jax.numpy as jnp
from jax import lax
from jax.experimental import pallas as pl
from jax.experimental.pallas import tpu as pltpu
from jax.experimental.pallas import tpu_sc as plsc

NUM_EMB = 256
IN_CH = 96
B = 4
H = 224
W = 224
IMG = H * W
NIMG = B * IN_CH
NTOT = NIMG * IMG
LANES = 16
NWORKERS = 32
HROWS = 112
NH = NIMG * 2
H_PER_W = NH // NWORKERS
NLSL = W // LANES


_RED_I = 4
_RED_J = 12
_RED_CB = IN_CH // _RED_J


def _reduce_body(x_ref, scale_ref, sc_ref, sum_ref, sq_ref, mx_ref):
    blk = x_ref[...]
    s = jnp.full((1, 1), jnp.sum(blk), dtype=jnp.float32)
    sq = jnp.full((1, 1), jnp.sum(blk * blk), dtype=jnp.float32)
    m = jnp.full((1, 1), jnp.max(jnp.abs(blk)), dtype=jnp.float32)

    first = jnp.logical_and(pl.program_id(0) == 0, pl.program_id(1) == 0)

    @pl.when(first)
    def _():
        zero = jnp.zeros((1, 1), jnp.float32)
        sum_ref[...] = zero
        sq_ref[...] = zero
        mx_ref[...] = zero

    sum_ref[...] += s
    sq_ref[...] += sq
    mx_ref[...] = jnp.maximum(mx_ref[...], m)

    last = jnp.logical_and(pl.program_id(0) == _RED_I - 1,
                           pl.program_id(1) == _RED_J - 1)

    @pl.when(last)
    def _():
        n = jnp.float32(NTOT)
        sv = sum_ref[...]
        sqv = sq_ref[...]
        mxv = mx_ref[...]
        var = (sqv - sv * sv / n) / (n - 1.0)
        std = jnp.sqrt(var)
        min_scale = 2.5 * 0.999 + std * 0.001
        max_scale = 3.5 * 0.999 + mxv * 0.001
        eps = 0.1 * (max_scale - min_scale)
        lo = jnp.broadcast_to(min_scale * (1.0 + eps), (8, 128))
        hi = jnp.broadcast_to(max_scale * (1.0 - eps), (8, 128))
        sc_ref[...] = jnp.minimum(jnp.maximum(scale_ref[...], lo), hi)


def _reductions_sc(x, scale8):
    out = pl.pallas_call(
        _reduce_body,
        grid=(_RED_I, _RED_J),
        in_specs=[
            pl.BlockSpec((1, _RED_CB, H, W), lambda i, j: (i, j, 0, 0)),
            pl.BlockSpec((8, 128), lambda i, j: (0, 0)),
        ],
        out_specs=[
            pl.BlockSpec((8, 128), lambda i, j: (0, 0)),
            pl.BlockSpec((1, 1), lambda i, j: (0, 0)),
            pl.BlockSpec((1, 1), lambda i, j: (0, 0)),
            pl.BlockSpec((1, 1), lambda i, j: (0, 0)),
        ],
        out_shape=[
            jax.ShapeDtypeStruct((8, 128), jnp.float32),
            jax.ShapeDtypeStruct((1, 1), jnp.float32),
            jax.ShapeDtypeStruct((1, 1), jnp.float32),
            jax.ShapeDtypeStruct((1, 1), jnp.float32),
        ],
    )(x, scale8)
    return out[0]


def _sc_compute_chunk(xbuf, tab_v, scv, offv):
    @plsc.parallel_loop(0, HROWS, step=1)
    def _(r):
        for t in range(NLSL):
            sl = pl.ds(t * LANES, LANES)
            xv = xbuf[r, sl]
            rr = xv / scv
            rr = jnp.minimum(jnp.maximum(rr, -1.0), 1.0)
            out01 = (rr + 1.0) * 0.5
            out3 = out01 * jnp.float32(NUM_EMB - 1)
            out4 = out3 + offv
            li = out4.astype(jnp.int32)
            lf = li.astype(jnp.float32)
            frac = out4 - lf
            ui = jnp.where(out4 > lf, li + 1, li)
            lv = plsc.load_gather(tab_v, [li])
            uv = plsc.load_gather(tab_v, [ui])
            res = frac * lv + (1.0 - frac) * uv
            xbuf[r, sl] = res


_NBUF = 3


def _sc_body(x_hbm, sc_hbm, emb_hbm, out_hbm, tab_v, scv_v, xb0, xb1, xb2,
             sin0, sin1, sin2, sout0, sout1, sout2):
    wid = lax.axis_index("s") * 2 + lax.axis_index("c")

    pltpu.sync_copy(emb_hbm, tab_v)
    pltpu.sync_copy(sc_hbm.at[0], scv_v)

    bufs = (xb0, xb1, xb2)
    sins = (sin0, sin1, sin2)
    souts = (sout0, sout1, sout2)
    h0 = wid * H_PER_W

    def addr(j):
        hm = h0 + j
        m = lax.div(hm, 2)
        half = lax.rem(hm, 2)
        bb = lax.div(m, IN_CH)
        cc = lax.rem(m, IN_CH)
        return bb, cc, half * HROWS

    def in_copy(j, b):
        bb, cc, r0 = addr(j)
        return pltpu.make_async_copy(
            x_hbm.at[bb, cc, pl.ds(r0, HROWS), :], bufs[b], sins[b])

    def out_copy(j, b):
        bb, cc, r0 = addr(j)
        return pltpu.make_async_copy(
            bufs[b], out_hbm.at[bb, cc, pl.ds(r0, HROWS), :], souts[b])

    in_copy(0, 0).start()
    in_copy(1, 1).start()

    def step(k, _):
        for b in range(_NBUF):
            j = k * _NBUF + b
            c = lax.rem(lax.div(h0 + j, 2), IN_CH)
            cvec = jnp.full((LANES,), c, dtype=jnp.int32)
            scv = plsc.load_gather(scv_v, [cvec])
            offv = jnp.full((LANES,), (c * NUM_EMB).astype(jnp.float32),
                            dtype=jnp.float32)

            in_copy(j, b).wait()
            _sc_compute_chunk(bufs[b], tab_v, scv, offv)
            out_copy(j, b).start()

            pb = (b + _NBUF - 1) % _NBUF
            jn = j + 2
            if b == 0:
                @pl.when(k > 0)
                def _():
                    out_copy(jn - _NBUF, pb).wait()

                in_copy(jn, pb).start()
            else:
                @pl.when(jn < H_PER_W)
                def _():
                    out_copy(jn - _NBUF, pb).wait()
                    in_copy(jn, pb).start()
        return 0

    lax.fori_loop(0, H_PER_W // _NBUF, step, 0)
    for b in range(_NBUF):
        out_copy(H_PER_W - _NBUF + b, b).wait()


def _sc_remap(x, sc8, emb_flat):
    mesh = plsc.VectorSubcoreMesh(core_axis_name="c", subcore_axis_name="s")
    fn = pl.kernel(
        _sc_body,
        out_type=jax.ShapeDtypeStruct((B, IN_CH, H, W), jnp.float32),
        mesh=mesh,
        compiler_params=pltpu.CompilerParams(
            needs_layout_passes=False, use_tc_tiling_on_sc=True),
        scratch_types=[
            pltpu.VMEM((NUM_EMB * IN_CH,), jnp.float32),
            pltpu.VMEM((128,), jnp.float32),
            pltpu.VMEM((HROWS, W), jnp.float32),
            pltpu.VMEM((HROWS, W), jnp.float32),
            pltpu.VMEM((HROWS, W), jnp.float32),
            pltpu.SemaphoreType.DMA,
            pltpu.SemaphoreType.DMA,
            pltpu.SemaphoreType.DMA,
            pltpu.SemaphoreType.DMA,
            pltpu.SemaphoreType.DMA,
            pltpu.SemaphoreType.DMA,
        ],
    )
    return fn(x, sc8, emb_flat)


def kernel(x, scale, emb_weight):
    scale8 = jnp.zeros((8, 128), jnp.float32).at[0, :IN_CH].set(
        scale.reshape(IN_CH))
    sc8 = _reductions_sc(x, scale8)
    return _sc_remap(x, sc8, emb_weight.reshape(NUM_EMB * IN_CH))


# speedup vs baseline: 1.3813x; 1.0194x over previous
import jax
import jax.numpy as jnp
from jax import lax
from jax.experimental import pallas as pl
from jax.experimental.pallas import tpu as pltpu
from jax.experimental.pallas import tpu_sc as plsc

NUM_EMB = 256
IN_CH = 96
B = 4
H = 224
W = 224
IMG = H * W
NIMG = B * IN_CH
NTOT = NIMG * IMG
LANES = 16
NWORKERS = 32
HROWS = 112
NH = NIMG * 2
H_PER_W = NH // NWORKERS
NLSL = W // LANES


_RED_I = 4
_RED_J = 12
_RED_CB = IN_CH // _RED_J


def _reduce_body(x_ref, scale_ref, sc_ref, sum_ref, sq_ref, mx_ref):
    blk = x_ref[...]
    s = jnp.full((1, 1), jnp.sum(blk), dtype=jnp.float32)
    sq = jnp.full((1, 1), jnp.sum(blk * blk), dtype=jnp.float32)
    m = jnp.full((1, 1), jnp.max(jnp.abs(blk)), dtype=jnp.float32)

    first = jnp.logical_and(pl.program_id(0) == 0, pl.program_id(1) == 0)

    @pl.when(first)
    def _():
        zero = jnp.zeros((1, 1), jnp.float32)
        sum_ref[...] = zero
        sq_ref[...] = zero
        mx_ref[...] = zero

    sum_ref[...] += s
    sq_ref[...] += sq
    mx_ref[...] = jnp.maximum(mx_ref[...], m)

    last = jnp.logical_and(pl.program_id(0) == _RED_I - 1,
                           pl.program_id(1) == _RED_J - 1)

    @pl.when(last)
    def _():
        n = jnp.float32(NTOT)
        sv = sum_ref[...]
        sqv = sq_ref[...]
        mxv = mx_ref[...]
        var = (sqv - sv * sv / n) / (n - 1.0)
        std = jnp.sqrt(var)
        min_scale = 2.5 * 0.999 + std * 0.001
        max_scale = 3.5 * 0.999 + mxv * 0.001
        eps = 0.1 * (max_scale - min_scale)
        lo = jnp.broadcast_to(min_scale * (1.0 + eps), (8, 128))
        hi = jnp.broadcast_to(max_scale * (1.0 - eps), (8, 128))
        sc_ref[...] = jnp.minimum(jnp.maximum(scale_ref[...], lo), hi)


def _reductions_sc(x, scale8):
    out = pl.pallas_call(
        _reduce_body,
        grid=(_RED_I, _RED_J),
        in_specs=[
            pl.BlockSpec((1, _RED_CB, H, W), lambda i, j: (i, j, 0, 0)),
            pl.BlockSpec((8, 128), lambda i, j: (0, 0)),
        ],
        out_specs=[
            pl.BlockSpec((8, 128), lambda i, j: (0, 0)),
            pl.BlockSpec((1, 1), lambda i, j: (0, 0)),
            pl.BlockSpec((1, 1), lambda i, j: (0, 0)),
            pl.BlockSpec((1, 1), lambda i, j: (0, 0)),
        ],
        out_shape=[
            jax.ShapeDtypeStruct((8, 128), jnp.float32),
            jax.ShapeDtypeStruct((1, 1), jnp.float32),
            jax.ShapeDtypeStruct((1, 1), jnp.float32),
            jax.ShapeDtypeStruct((1, 1), jnp.float32),
        ],
    )(x, scale8)
    return out[0]


def _sc_compute_chunk(xbuf, tab_v, scv, offv):
    @plsc.parallel_loop(0, HROWS, step=1, unroll=2)
    def _(r):
        for t in range(NLSL):
            sl = pl.ds(t * LANES, LANES)
            xv = xbuf[r, sl]
            rr = xv / scv
            rr = jnp.minimum(jnp.maximum(rr, -1.0), 1.0)
            out01 = (rr + 1.0) * 0.5
            out3 = out01 * jnp.float32(NUM_EMB - 1)
            out4 = out3 + offv
            li = out4.astype(jnp.int32)
            lf = li.astype(jnp.float32)
            frac = out4 - lf
            ui = jnp.where(out4 > lf, li + 1, li)
            lv = plsc.load_gather(tab_v, [li])
            uv = plsc.load_gather(tab_v, [ui])
            res = frac * lv + (1.0 - frac) * uv
            xbuf[r, sl] = res


_NBUF = 3


def _sc_body(x_hbm, sc_hbm, emb_hbm, out_hbm, tab_v, scv_v, xb0, xb1, xb2,
             sin0, sin1, sin2, sout0, sout1, sout2):
    wid = lax.axis_index("s") * 2 + lax.axis_index("c")

    pltpu.sync_copy(emb_hbm, tab_v)
    pltpu.sync_copy(sc_hbm.at[0], scv_v)

    bufs = (xb0, xb1, xb2)
    sins = (sin0, sin1, sin2)
    souts = (sout0, sout1, sout2)
    h0 = wid * H_PER_W

    def addr(j):
        hm = h0 + j
        m = lax.div(hm, 2)
        half = lax.rem(hm, 2)
        bb = lax.div(m, IN_CH)
        cc = lax.rem(m, IN_CH)
        return bb, cc, half * HROWS

    def in_copy(j, b):
        bb, cc, r0 = addr(j)
        return pltpu.make_async_copy(
            x_hbm.at[bb, cc, pl.ds(r0, HROWS), :], bufs[b], sins[b])

    def out_copy(j, b):
        bb, cc, r0 = addr(j)
        return pltpu.make_async_copy(
            bufs[b], out_hbm.at[bb, cc, pl.ds(r0, HROWS), :], souts[b])

    in_copy(0, 0).start()
    in_copy(1, 1).start()

    def step(k, _):
        for b in range(_NBUF):
            j = k * _NBUF + b
            c = lax.rem(lax.div(h0 + j, 2), IN_CH)
            cvec = jnp.full((LANES,), c, dtype=jnp.int32)
            scv = plsc.load_gather(scv_v, [cvec])
            offv = jnp.full((LANES,), (c * NUM_EMB).astype(jnp.float32),
                            dtype=jnp.float32)

            in_copy(j, b).wait()
            _sc_compute_chunk(bufs[b], tab_v, scv, offv)
            out_copy(j, b).start()

            pb = (b + _NBUF - 1) % _NBUF
            jn = j + 2
            if b == 0:
                @pl.when(k > 0)
                def _():
                    out_copy(jn - _NBUF, pb).wait()

                in_copy(jn, pb).start()
            else:
                @pl.when(jn < H_PER_W)
                def _():
                    out_copy(jn - _NBUF, pb).wait()
                    in_copy(jn, pb).start()
        return 0

    lax.fori_loop(0, H_PER_W // _NBUF, step, 0)
    for b in range(_NBUF):
        out_copy(H_PER_W - _NBUF + b, b).wait()


def _sc_remap(x, sc8, emb_flat):
    mesh = plsc.VectorSubcoreMesh(core_axis_name="c", subcore_axis_name="s")
    fn = pl.kernel(
        _sc_body,
        out_type=jax.ShapeDtypeStruct((B, IN_CH, H, W), jnp.float32),
        mesh=mesh,
        compiler_params=pltpu.CompilerParams(
            needs_layout_passes=False, use_tc_tiling_on_sc=True),
        scratch_types=[
            pltpu.VMEM((NUM_EMB * IN_CH,), jnp.float32),
            pltpu.VMEM((128,), jnp.float32),
            pltpu.VMEM((HROWS, W), jnp.float32),
            pltpu.VMEM((HROWS, W), jnp.float32),
            pltpu.VMEM((HROWS, W), jnp.float32),
            pltpu.SemaphoreType.DMA,
            pltpu.SemaphoreType.DMA,
            pltpu.SemaphoreType.DMA,
            pltpu.SemaphoreType.DMA,
            pltpu.SemaphoreType.DMA,
            pltpu.SemaphoreType.DMA,
        ],
    )
    return fn(x, sc8, emb_flat)


def kernel(x, scale, emb_weight):
    scale8 = jnp.zeros((8, 128), jnp.float32).at[0, :IN_CH].set(
        scale.reshape(IN_CH))
    sc8 = _reductions_sc(x, scale8)
    return _sc_remap(x, sc8, emb_weight.reshape(NUM_EMB * IN_CH))
